# Initial kernel scaffold; baseline (speedup 1.0000x reference)
import jax
import jax.numpy as jnp
from jax.experimental import pallas as pl


def kernel(x, mask, X_features, W1, b1, Wo, bo):
    raise NotImplementedError("write your pallas kernel here")


# speedup vs baseline: 1.6868x; 1.6868x over previous
import functools

import jax
import jax.numpy as jnp
from jax import lax
from jax.experimental import pallas as pl
from jax.experimental.pallas import tpu as pltpu
from jax.experimental.pallas import tpu_sc as plsc

B, S, D = 16, 4096, 128
C = 64
M = 65536
N = B * S
GRID = 0.01
MT = M + 1024

BN = 1024
NSUB = 16
TPW = N // NSUB
CH = 16


def _ka_body(x_ref, xf_ref, mv_ref, w1_ref, b1_ref, h_ref, seg_ref):
    i = pl.program_id(0)
    x = x_ref[...]
    h_ref[...] = jnp.maximum(jnp.dot(x, w1_ref[...],
                                     preferred_element_type=jnp.float32)
                             + b1_ref[...], 0.0)
    v0 = jnp.floor(xf_ref[:, 2:3] / GRID).astype(jnp.int32)
    v1 = jnp.floor(xf_ref[:, 3:4] / GRID).astype(jnp.int32)
    v2 = jnp.floor(xf_ref[:, 4:5] / GRID).astype(jnp.int32)
    tok = i * BN + lax.broadcasted_iota(jnp.int32, (BN, 1), 0)
    batch = lax.shift_right_logical(tok, 12)
    hcode = (batch * 73856093 + v0 * 19349663
             + v1 * 83492791 + v2 * 49979687)
    seg = jnp.bitwise_and(hcode, M - 1)
    trash = M + jnp.bitwise_and(tok, 1023)
    seg_ref[...] = jnp.where(mv_ref[...] > 0, seg, trash)


def _run_a(x2, xf2, mvalid, W1, b1):
    return pl.pallas_call(
        _ka_body,
        grid=(N // BN,),
        in_specs=[
            pl.BlockSpec((BN, D), lambda i: (i, 0)),
            pl.BlockSpec((BN, 8), lambda i: (i, 0)),
            pl.BlockSpec((BN, 1), lambda i: (i, 0)),
            pl.BlockSpec((D, C), lambda i: (0, 0)),
            pl.BlockSpec((1, C), lambda i: (0, 0)),
        ],
        out_specs=[
            pl.BlockSpec((BN, C), lambda i: (i, 0)),
            pl.BlockSpec((BN, 1), lambda i: (i, 0)),
        ],
        out_shape=[
            jax.ShapeDtypeStruct((N, C), jnp.float32),
            jax.ShapeDtypeStruct((N, 1), jnp.int32),
        ],
    )(x2, xf2, mvalid, W1, b1)


def _kb_body(h_hbm, seg_hbm, g_hbm, cnt_hbm,
             segv, row0, row1, cbuf, zbuf, z1buf, onesv, sem0, sem1,
             table, ctable):
    c = lax.axis_index("c")
    s = lax.axis_index("s")
    tok0 = s * TPW
    bufs = (row0, row1)
    sems = (sem0, sem1)
    QCH = 1024
    NQ = TPW // QCH

    zv = jnp.zeros((16,), jnp.float32)

    def _z2(j, _):
        zbuf[j] = zv
        return 0
    lax.fori_loop(0, 256, _z2, 0)

    def _z1(j, _):
        z1buf[pl.ds(j * 16, 16)] = zv
        return 0
    lax.fori_loop(0, 64, _z1, 0)

    ov = jnp.ones((16,), jnp.float32)
    for j in range(8):
        onesv[pl.ds(j * 16, 16)] = ov

    pltpu.sync_copy(seg_hbm.at[pl.ds(s * 32, 32)], segv)

    for r in range(2):
        ch0 = c * 32 + r * CH

        def _read(q):
            return pltpu.async_copy(
                h_hbm.at[pl.ds(tok0 + q * QCH, QCH), pl.ds(ch0, CH)],
                bufs[q % 2], sems[q % 2])

        cps = {0: _read(0)}

        for j in range(16):
            pltpu.sync_copy(zbuf, table.at[pl.ds(s * 4096 + j * 256, 256)])

        if r == 0:
            @pl.when(c == 0)
            def _():
                for j in range(4):
                    pltpu.sync_copy(
                        z1buf, ctable.at[pl.ds(s * 4096 + j * 1024, 1024)])

        plsc.subcore_barrier()

        for q in range(NQ):
            cps[q].wait()
            if q + 1 < NQ:
                cps[q + 1] = _read(q + 1)
            for k in range(8):
                pltpu.sync_copy(bufs[q % 2].at[pl.ds(k * 128, 128)],
                                table.at[segv.at[q * 8 + k]], add=True)

        if r == 0:
            @pl.when(c == 0)
            def _():
                def _cscatter(k, _):
                    pltpu.sync_copy(onesv, ctable.at[segv.at[k]], add=True)
                    return 0
                lax.fori_loop(0, 32, _cscatter, 0)

        plsc.subcore_barrier()

        wr = {}
        for q in range(NQ):
            if q >= 2:
                wr[q - 2].wait()
            for k in range(8):
                pltpu.sync_copy(table.at[segv.at[q * 8 + k]],
                                bufs[q % 2].at[pl.ds(k * 128, 128)])
            wr[q] = pltpu.async_copy(
                bufs[q % 2],
                g_hbm.at[pl.ds(tok0 + q * QCH, QCH), pl.ds(ch0, CH)],
                sems[q % 2])
        wr[NQ - 2].wait()
        wr[NQ - 1].wait()

        if r == 0:
            @pl.when(c == 0)
            def _():
                def _cgather(k, _):
                    pltpu.sync_copy(ctable.at[segv.at[k]],
                                    cbuf.at[pl.ds(k * 128, 128)])
                    return 0
                lax.fori_loop(0, 32, _cgather, 0)
                pltpu.sync_copy(cbuf, cnt_hbm.at[pl.ds(tok0, TPW)])

        plsc.subcore_barrier()


def _run_b(h, seg2):
    kb = functools.partial(
        pl.kernel,
        out_type=[
            jax.ShapeDtypeStruct((N, C), jnp.float32),
            jax.ShapeDtypeStruct((N,), jnp.float32),
        ],
        mesh=plsc.VectorSubcoreMesh(core_axis_name="c", subcore_axis_name="s"),
        compiler_params=pltpu.CompilerParams(use_tc_tiling_on_sc=False),
        scratch_types=[
            pltpu.VMEM((32, 128), jnp.int32),
            pltpu.VMEM((1024, CH), jnp.float32),
            pltpu.VMEM((1024, CH), jnp.float32),
            pltpu.VMEM((TPW,), jnp.float32),
            pltpu.VMEM((256, CH), jnp.float32),
            pltpu.VMEM((1024,), jnp.float32),
            pltpu.VMEM((128,), jnp.float32),
            pltpu.SemaphoreType.DMA,
            pltpu.SemaphoreType.DMA,
            pltpu.VMEM_SHARED((MT, CH), jnp.float32),
            pltpu.VMEM_SHARED((MT,), jnp.float32),
        ],
    )(_kb_body)
    return kb(h, seg2)


def _kc_body(h_ref, g_ref, cnt_ref, mv_ref, wo_ref, bo_ref, o_ref):
    t = h_ref[...] + g_ref[...] / jnp.maximum(cnt_ref[...], 1.0)
    y = jnp.dot(t, wo_ref[...], preferred_element_type=jnp.float32) + bo_ref[...]
    o_ref[...] = jnp.where(mv_ref[...] > 0, y, 0.0)


def _run_c(h, g, cnt, mvalid, Wo, bo):
    return pl.pallas_call(
        _kc_body,
        grid=(N // BN,),
        in_specs=[
            pl.BlockSpec((BN, C), lambda i: (i, 0)),
            pl.BlockSpec((BN, C), lambda i: (i, 0)),
            pl.BlockSpec((BN, 1), lambda i: (i, 0)),
            pl.BlockSpec((BN, 1), lambda i: (i, 0)),
            pl.BlockSpec((C, D), lambda i: (0, 0)),
            pl.BlockSpec((1, D), lambda i: (0, 0)),
        ],
        out_specs=pl.BlockSpec((BN, D), lambda i: (i, 0)),
        out_shape=jax.ShapeDtypeStruct((N, D), jnp.float32),
    )(h, g, cnt, mvalid, Wo, bo)


def kernel(x, mask, X_features, W1, b1, Wo, bo):
    x2 = x.reshape(N, D).astype(jnp.float32)
    xf2 = X_features.reshape(N, 8).astype(jnp.float32)
    mvalid = mask.reshape(N, 1).astype(jnp.int32)

    h, seg = _run_a(x2, xf2, mvalid, W1.astype(jnp.float32),
                    b1.reshape(1, C).astype(jnp.float32))
    g, cnt = _run_b(h, seg.reshape(512, 128))
    out = _run_c(h, g, cnt.reshape(N, 1), mvalid,
                 Wo.astype(jnp.float32), bo.reshape(1, D).astype(jnp.float32))
    return out.reshape(B, S, D).astype(x.dtype)


# speedup vs baseline: 3.5152x; 2.0839x over previous
import functools

import jax
import jax.numpy as jnp
from jax import lax
from jax.experimental import pallas as pl
from jax.experimental.pallas import tpu as pltpu
from jax.experimental.pallas import tpu_sc as plsc

B, S, D = 16, 4096, 128
C = 64
M = 65536
N = B * S
GRID = 0.01
NTRASH = 1024
MT = M + NTRASH

NSUB = 16
TPW = N // NSUB
CH = 16
H = N // 2
BR = 1024


def _ka_body(xl_ref, xr_ref, xtl_ref, xtr_ref, mvl_ref, mvr_ref,
             w1_ref, b1_ref, h2_ref, seg_ref):
    i = pl.program_id(0)
    w1 = w1_ref[...]
    b1 = b1_ref[...]
    hl = jnp.maximum(jnp.dot(xl_ref[...], w1,
                             preferred_element_type=jnp.float32) + b1, 0.0)
    hr = jnp.maximum(jnp.dot(xr_ref[...], w1,
                             preferred_element_type=jnp.float32) + b1, 0.0)
    h2_ref[...] = jnp.concatenate([hl, hr], axis=1)

    def _seg(xt_blk, mv_blk, tok0):
        v0 = jnp.floor(xt_blk[0] / GRID).astype(jnp.int32)
        v1 = jnp.floor(xt_blk[1] / GRID).astype(jnp.int32)
        v2 = jnp.floor(xt_blk[2] / GRID).astype(jnp.int32)
        tok = (tok0
               + lax.broadcasted_iota(jnp.int32, (8, 128), 0) * 128
               + lax.broadcasted_iota(jnp.int32, (8, 128), 1))
        batch = lax.shift_right_logical(tok, 12)
        hcode = (batch * 73856093 + v0 * 19349663
                 + v1 * 83492791 + v2 * 49979687)
        seg = jnp.bitwise_and(hcode, M - 1)
        trash = M + jnp.bitwise_and(tok, NTRASH - 1)
        return jnp.where(mv_blk > 0, seg, trash)

    seg_ref[0] = _seg(xtl_ref[...], mvl_ref[...], i * BR)
    seg_ref[1] = _seg(xtr_ref[...], mvr_ref[...], H + i * BR)


def _run_a(x2, xt3, mv, W1, b1):
    return pl.pallas_call(
        _ka_body,
        grid=(H // BR,),
        in_specs=[
            pl.BlockSpec((BR, D), lambda i: (i, 0)),
            pl.BlockSpec((BR, D), lambda i: (i + 32, 0)),
            pl.BlockSpec((3, 8, 128), lambda i: (0, i, 0)),
            pl.BlockSpec((3, 8, 128), lambda i: (0, i + 32, 0)),
            pl.BlockSpec((8, 128), lambda i: (i, 0)),
            pl.BlockSpec((8, 128), lambda i: (i + 32, 0)),
            pl.BlockSpec((D, C), lambda i: (0, 0)),
            pl.BlockSpec((1, C), lambda i: (0, 0)),
        ],
        out_specs=[
            pl.BlockSpec((BR, 2 * C), lambda i: (i, 0)),
            pl.BlockSpec((2, 8, 128), lambda i: (0, i, 0)),
        ],
        out_shape=[
            jax.ShapeDtypeStruct((H, 2 * C), jnp.float32),
            jax.ShapeDtypeStruct((2, 256, 128), jnp.int32),
        ],
    )(x2, x2, xt3, xt3, mv, mv, W1, b1)


def _kb_body(h_hbm, seg_hbm, g_hbm, cnt_hbm,
             segv, row0, row1, cbuf, zbuf, z1buf, onesv, sem0, sem1,
             table, ctable):
    c = lax.axis_index("c")
    s = lax.axis_index("s")
    tok0 = s * TPW
    rowbase = (s % 8) * TPW
    halfoff = (s // 8) * C
    bufs = (row0, row1)
    sems = (sem0, sem1)
    QCH = 1024
    NQ = TPW // QCH

    zv = jnp.zeros((16,), jnp.float32)

    def _z2(j, _):
        zbuf[j] = zv
        return 0
    lax.fori_loop(0, 256, _z2, 0)

    def _z1(j, _):
        z1buf[pl.ds(j * 16, 16)] = zv
        return 0
    lax.fori_loop(0, 64, _z1, 0)

    ov = jnp.ones((16,), jnp.float32)
    for j in range(8):
        onesv[pl.ds(j * 16, 16)] = ov

    pltpu.sync_copy(seg_hbm.at[pl.ds(s * 32, 32)], segv)

    for r in range(2):
        ch0 = halfoff + c * 32 + r * CH

        def _read(q):
            return pltpu.async_copy(
                h_hbm.at[pl.ds(rowbase + q * QCH, QCH), pl.ds(ch0, CH)],
                bufs[q % 2], sems[q % 2])

        cps = {0: _read(0)}

        for j in range(16):
            pltpu.sync_copy(zbuf, table.at[pl.ds(s * 4096 + j * 256, 256)])
        pltpu.sync_copy(zbuf.at[pl.ds(0, 64)],
                        table.at[pl.ds(M + s * 64, 64)])

        if r == 0:
            @pl.when(c == 0)
            def _():
                for j in range(4):
                    pltpu.sync_copy(
                        z1buf, ctable.at[pl.ds(s * 4096 + j * 1024, 1024)])
                pltpu.sync_copy(z1buf.at[pl.ds(0, 64)],
                                ctable.at[pl.ds(M + s * 64, 64)])

        plsc.subcore_barrier()

        for q in range(NQ):
            cps[q].wait()
            if q + 1 < NQ:
                cps[q + 1] = _read(q + 1)
            for k in range(8):
                pltpu.sync_copy(bufs[q % 2].at[pl.ds(k * 128, 128)],
                                table.at[segv.at[q * 8 + k]], add=True)

        if r == 0:
            @pl.when(c == 0)
            def _():
                def _cscatter(k, _):
                    pltpu.sync_copy(onesv, ctable.at[segv.at[k]], add=True)
                    return 0
                lax.fori_loop(0, 32, _cscatter, 0)

        plsc.subcore_barrier()

        wr = {}
        for q in range(NQ):
            if q >= 2:
                wr[q - 2].wait()
            for k in range(8):
                pltpu.sync_copy(table.at[segv.at[q * 8 + k]],
                                bufs[q % 2].at[pl.ds(k * 128, 128)])
            wr[q] = pltpu.async_copy(
                bufs[q % 2],
                g_hbm.at[pl.ds(rowbase + q * QCH, QCH), pl.ds(ch0, CH)],
                sems[q % 2])
        wr[NQ - 2].wait()
        wr[NQ - 1].wait()

        if r == 0:
            @pl.when(c == 0)
            def _():
                def _cgather(k, _):
                    pltpu.sync_copy(ctable.at[segv.at[k]],
                                    cbuf.at[pl.ds(k * 128, 128)])
                    return 0
                lax.fori_loop(0, 32, _cgather, 0)
                pltpu.sync_copy(cbuf, cnt_hbm.at[pl.ds(tok0, TPW)])

        plsc.subcore_barrier()


def _run_b(h2, seg2):
    kb = functools.partial(
        pl.kernel,
        out_type=[
            jax.ShapeDtypeStruct((H, 2 * C), jnp.float32),
            jax.ShapeDtypeStruct((N,), jnp.float32),
        ],
        mesh=plsc.VectorSubcoreMesh(core_axis_name="c", subcore_axis_name="s"),
        compiler_params=pltpu.CompilerParams(use_tc_tiling_on_sc=False),
        scratch_types=[
            pltpu.VMEM((32, 128), jnp.int32),
            pltpu.VMEM((1024, CH), jnp.float32),
            pltpu.VMEM((1024, CH), jnp.float32),
            pltpu.VMEM((TPW,), jnp.float32),
            pltpu.VMEM((256, CH), jnp.float32),
            pltpu.VMEM((1024,), jnp.float32),
            pltpu.VMEM((128,), jnp.float32),
            pltpu.SemaphoreType.DMA,
            pltpu.SemaphoreType.DMA,
            pltpu.VMEM_SHARED((MT, CH), jnp.float32),
            pltpu.VMEM_SHARED((MT,), jnp.float32),
        ],
    )(_kb_body)
    return kb(h2, seg2)


def _kc_body(h2_ref, g2_ref, cl_ref, cr_ref, mvl_ref, mvr_ref,
             wo_ref, bo_ref, o_ref):
    wo = wo_ref[...]
    bo = bo_ref[...]
    rsel = (lax.broadcasted_iota(jnp.int32, (BR, 8), 0) // 128
            == lax.broadcasted_iota(jnp.int32, (BR, 8), 1)
            ).astype(jnp.float32)
    lsel = (lax.broadcasted_iota(jnp.int32, (BR, 128), 0) % 128
            == lax.broadcasted_iota(jnp.int32, (BR, 128), 1)
            ).astype(jnp.float32)

    def _cols(blk4):
        expanded = jnp.dot(rsel, blk4, preferred_element_type=jnp.float32)
        return jnp.sum(expanded * lsel, axis=1, keepdims=True)

    h2 = h2_ref[...]
    g2 = g2_ref[...]

    def _half(hs, gs, cnt4, mv4):
        a4 = (mv4 > 0).astype(jnp.float32)
        rc4 = a4 / jnp.maximum(cnt4, 1.0)
        a_col = _cols(a4)
        b_col = _cols(rc4)
        t = hs + gs * b_col
        y = jnp.dot(t, wo, preferred_element_type=jnp.float32) + bo
        return a_col * y

    yl = _half(h2[:, :C], g2[:, :C], cl_ref[...], mvl_ref[...])
    yr = _half(h2[:, C:], g2[:, C:], cr_ref[...], mvr_ref[...])
    o_ref[0] = yl
    o_ref[1] = yr


def _run_c(h2, g2, cnt2, mv, Wo, bo):
    return pl.pallas_call(
        _kc_body,
        grid=(H // BR,),
        in_specs=[
            pl.BlockSpec((BR, 2 * C), lambda i: (i, 0)),
            pl.BlockSpec((BR, 2 * C), lambda i: (i, 0)),
            pl.BlockSpec((8, 128), lambda i: (i, 0)),
            pl.BlockSpec((8, 128), lambda i: (i + 32, 0)),
            pl.BlockSpec((8, 128), lambda i: (i, 0)),
            pl.BlockSpec((8, 128), lambda i: (i + 32, 0)),
            pl.BlockSpec((C, D), lambda i: (0, 0)),
            pl.BlockSpec((1, D), lambda i: (0, 0)),
        ],
        out_specs=pl.BlockSpec((2, BR, D), lambda i: (0, i, 0)),
        out_shape=jax.ShapeDtypeStruct((2, H, D), jnp.float32),
    )(h2, g2, cnt2, cnt2, mv, mv, Wo, bo)


def kernel(x, mask, X_features, W1, b1, Wo, bo):
    x2 = x.reshape(N, D).astype(jnp.float32)
    xt3 = (X_features.reshape(N, 8)[:, 2:5].astype(jnp.float32)
           .T.reshape(3, 512, 128))
    mv = mask.reshape(512, 128).astype(jnp.int32)

    h2, seg = _run_a(x2, xt3, mv, W1.astype(jnp.float32),
                     b1.reshape(1, C).astype(jnp.float32))
    g2, cnt = _run_b(h2, seg.reshape(512, 128))
    out = _run_c(h2, g2, cnt.reshape(512, 128), mv,
                 Wo.astype(jnp.float32), bo.reshape(1, D).astype(jnp.float32))
    return out.reshape(N, D).reshape(B, S, D).astype(x.dtype)


# speedup vs baseline: 3.8584x; 1.0976x over previous
import functools

import jax
import jax.numpy as jnp
from jax import lax
from jax.experimental import pallas as pl
from jax.experimental.pallas import tpu as pltpu
from jax.experimental.pallas import tpu_sc as plsc

B, S, D = 16, 4096, 128
C = 64
M = 65536
N = B * S
GRID = 0.01
NTRASH = 1024
MT = M + NTRASH

NSUB = 16
TPW = N // NSUB
CH = 16
H = N // 2
BR = 1024


def _ka_body(xl_ref, xr_ref, xtl_ref, xtr_ref, mvl_ref, mvr_ref,
             w1_ref, b1_ref, h2_ref, seg_ref):
    i = pl.program_id(0)
    w1 = w1_ref[...]
    b1 = b1_ref[...]
    hl = jnp.maximum(jnp.dot(xl_ref[...], w1,
                             preferred_element_type=jnp.float32) + b1, 0.0)
    hr = jnp.maximum(jnp.dot(xr_ref[...], w1,
                             preferred_element_type=jnp.float32) + b1, 0.0)
    h2_ref[...] = jnp.concatenate([hl, hr], axis=1)

    def _seg(xt_blk, mv_blk, tok0):
        v0 = jnp.floor(xt_blk[0] / GRID).astype(jnp.int32)
        v1 = jnp.floor(xt_blk[1] / GRID).astype(jnp.int32)
        v2 = jnp.floor(xt_blk[2] / GRID).astype(jnp.int32)
        tok = (tok0
               + lax.broadcasted_iota(jnp.int32, (8, 128), 0) * 128
               + lax.broadcasted_iota(jnp.int32, (8, 128), 1))
        batch = lax.shift_right_logical(tok, 12)
        hcode = (batch * 73856093 + v0 * 19349663
                 + v1 * 83492791 + v2 * 49979687)
        seg = jnp.bitwise_and(hcode, M - 1)
        trash = M + jnp.bitwise_and(tok, NTRASH - 1)
        return jnp.where(mv_blk > 0, seg, trash)

    seg_ref[0] = _seg(xtl_ref[...], mvl_ref[...], i * BR)
    seg_ref[1] = _seg(xtr_ref[...], mvr_ref[...], H + i * BR)


def _run_a(x2, xt3, mv, W1, b1):
    return pl.pallas_call(
        _ka_body,
        grid=(H // BR,),
        in_specs=[
            pl.BlockSpec((BR, D), lambda i: (i, 0)),
            pl.BlockSpec((BR, D), lambda i: (i + 32, 0)),
            pl.BlockSpec((3, 8, 128), lambda i: (0, i, 0)),
            pl.BlockSpec((3, 8, 128), lambda i: (0, i + 32, 0)),
            pl.BlockSpec((8, 128), lambda i: (i, 0)),
            pl.BlockSpec((8, 128), lambda i: (i + 32, 0)),
            pl.BlockSpec((D, C), lambda i: (0, 0)),
            pl.BlockSpec((1, C), lambda i: (0, 0)),
        ],
        out_specs=[
            pl.BlockSpec((BR, 2 * C), lambda i: (i, 0)),
            pl.BlockSpec((2, 8, 128), lambda i: (0, i, 0)),
        ],
        out_shape=[
            jax.ShapeDtypeStruct((H, 2 * C), jnp.float32),
            jax.ShapeDtypeStruct((2, 256, 128), jnp.int32),
        ],
    )(x2, x2, xt3, xt3, mv, mv, W1, b1)


def _kb_body(h_hbm, seg_hbm, g_hbm, cnt_hbm,
             segv, row0, row1, cbuf, zbuf, z1buf, onesv, sem0, sem1,
             sem_sc, sem_c, table, ctable):
    c = lax.axis_index("c")
    s = lax.axis_index("s")
    tok0 = s * TPW
    rowbase = (s % 8) * TPW
    halfoff = (s // 8) * C
    bufs = (row0, row1)
    sems = (sem0, sem1)
    QCH = 1024
    NQ = TPW // QCH

    zv = jnp.zeros((16,), jnp.float32)

    def _z2(j, _):
        zbuf[j] = zv
        return 0
    lax.fori_loop(0, 256, _z2, 0)

    def _z1(j, _):
        z1buf[pl.ds(j * 16, 16)] = zv
        return 0
    lax.fori_loop(0, 64, _z1, 0)

    ov = jnp.ones((16,), jnp.float32)
    for j in range(8):
        onesv[pl.ds(j * 16, 16)] = ov

    pltpu.sync_copy(seg_hbm.at[pl.ds(s * 32, 32)], segv)

    for r in range(2):
        ch0 = halfoff + c * 32 + r * CH

        def _read(q):
            return pltpu.async_copy(
                h_hbm.at[pl.ds(rowbase + q * QCH, QCH), pl.ds(ch0, CH)],
                bufs[q % 2], sems[q % 2])

        cps = {0: _read(0), 1: _read(1)}

        for j in range(16):
            pltpu.sync_copy(zbuf, table.at[pl.ds(s * 4096 + j * 256, 256)])
        pltpu.sync_copy(zbuf.at[pl.ds(0, 64)],
                        table.at[pl.ds(M + s * 64, 64)])

        if r == 0:
            @pl.when(c == 0)
            def _():
                for j in range(4):
                    pltpu.sync_copy(
                        z1buf, ctable.at[pl.ds(s * 4096 + j * 1024, 1024)])
                pltpu.sync_copy(z1buf.at[pl.ds(0, 64)],
                                ctable.at[pl.ds(M + s * 64, 64)])

        plsc.subcore_barrier()

        for q in range(NQ):
            cps[q].wait()
            for k in range(8):
                pltpu.async_copy(bufs[q % 2].at[pl.ds(k * 128, 128)],
                                 table.at[segv.at[q * 8 + k]], sem_sc,
                                 add=True)
            for k in range(8):
                pltpu.make_async_copy(
                    bufs[q % 2].at[pl.ds(k * 128, 128)],
                    table.at[segv.at[q * 8 + k]], sem_sc).wait()
            if q + 2 < NQ:
                cps[q + 2] = _read(q + 2)

        if r == 0:
            @pl.when(c == 0)
            def _():
                def _cscatter(k, _):
                    pltpu.async_copy(onesv, ctable.at[segv.at[k]], sem_c,
                                     add=True)
                    return 0
                lax.fori_loop(0, 32, _cscatter, 0)

                def _cdrain(k, _):
                    pltpu.make_async_copy(onesv, ctable.at[segv.at[k]],
                                          sem_c).wait()
                    return 0
                lax.fori_loop(0, 32, _cdrain, 0)

        plsc.subcore_barrier()

        wr = {}
        for q in range(NQ):
            if q >= 2:
                wr[q - 2].wait()
            for k in range(8):
                pltpu.async_copy(table.at[segv.at[q * 8 + k]],
                                 bufs[q % 2].at[pl.ds(k * 128, 128)], sem_sc)
            for k in range(8):
                pltpu.make_async_copy(
                    table.at[segv.at[q * 8 + k]],
                    bufs[q % 2].at[pl.ds(k * 128, 128)], sem_sc).wait()
            wr[q] = pltpu.async_copy(
                bufs[q % 2],
                g_hbm.at[pl.ds(rowbase + q * QCH, QCH), pl.ds(ch0, CH)],
                sems[q % 2])
        wr[NQ - 2].wait()
        wr[NQ - 1].wait()

        if r == 0:
            @pl.when(c == 0)
            def _():
                def _cgfire(k, _):
                    pltpu.async_copy(ctable.at[segv.at[k]],
                                     cbuf.at[pl.ds(k * 128, 128)], sem_c)
                    return 0
                lax.fori_loop(0, 32, _cgfire, 0)

                def _cgdrain(k, _):
                    pltpu.make_async_copy(ctable.at[segv.at[k]],
                                          cbuf.at[pl.ds(k * 128, 128)],
                                          sem_c).wait()
                    return 0
                lax.fori_loop(0, 32, _cgdrain, 0)
                pltpu.sync_copy(cbuf, cnt_hbm.at[pl.ds(tok0, TPW)])

        plsc.subcore_barrier()


def _run_b(h2, seg2):
    kb = functools.partial(
        pl.kernel,
        out_type=[
            jax.ShapeDtypeStruct((H, 2 * C), jnp.float32),
            jax.ShapeDtypeStruct((N,), jnp.float32),
        ],
        mesh=plsc.VectorSubcoreMesh(core_axis_name="c", subcore_axis_name="s"),
        compiler_params=pltpu.CompilerParams(use_tc_tiling_on_sc=False),
        scratch_types=[
            pltpu.VMEM((32, 128), jnp.int32),
            pltpu.VMEM((1024, CH), jnp.float32),
            pltpu.VMEM((1024, CH), jnp.float32),
            pltpu.VMEM((TPW,), jnp.float32),
            pltpu.VMEM((256, CH), jnp.float32),
            pltpu.VMEM((1024,), jnp.float32),
            pltpu.VMEM((128,), jnp.float32),
            pltpu.SemaphoreType.DMA,
            pltpu.SemaphoreType.DMA,
            pltpu.SemaphoreType.DMA,
            pltpu.SemaphoreType.DMA,
            pltpu.VMEM_SHARED((MT, CH), jnp.float32),
            pltpu.VMEM_SHARED((MT,), jnp.float32),
        ],
    )(_kb_body)
    return kb(h2, seg2)


def _kc_body(h2_ref, g2_ref, cl_ref, cr_ref, mvl_ref, mvr_ref,
             wo_ref, bo_ref, o_ref):
    wo = wo_ref[...]
    bo = bo_ref[...]
    rsel = (lax.broadcasted_iota(jnp.int32, (BR, 8), 0) // 128
            == lax.broadcasted_iota(jnp.int32, (BR, 8), 1)
            ).astype(jnp.float32)
    lsel = (lax.broadcasted_iota(jnp.int32, (BR, 128), 0) % 128
            == lax.broadcasted_iota(jnp.int32, (BR, 128), 1)
            ).astype(jnp.float32)

    def _cols(blk4):
        expanded = jnp.dot(rsel, blk4, preferred_element_type=jnp.float32)
        return jnp.sum(expanded * lsel, axis=1, keepdims=True)

    h2 = h2_ref[...]
    g2 = g2_ref[...]

    def _half(hs, gs, cnt4, mv4):
        a4 = (mv4 > 0).astype(jnp.float32)
        rc4 = a4 / jnp.maximum(cnt4, 1.0)
        a_col = _cols(a4)
        b_col = _cols(rc4)
        t = hs + gs * b_col
        y = jnp.dot(t, wo, preferred_element_type=jnp.float32) + bo
        return a_col * y

    yl = _half(h2[:, :C], g2[:, :C], cl_ref[...], mvl_ref[...])
    yr = _half(h2[:, C:], g2[:, C:], cr_ref[...], mvr_ref[...])
    o_ref[0] = yl
    o_ref[1] = yr


def _run_c(h2, g2, cnt2, mv, Wo, bo):
    return pl.pallas_call(
        _kc_body,
        grid=(H // BR,),
        in_specs=[
            pl.BlockSpec((BR, 2 * C), lambda i: (i, 0)),
            pl.BlockSpec((BR, 2 * C), lambda i: (i, 0)),
            pl.BlockSpec((8, 128), lambda i: (i, 0)),
            pl.BlockSpec((8, 128), lambda i: (i + 32, 0)),
            pl.BlockSpec((8, 128), lambda i: (i, 0)),
            pl.BlockSpec((8, 128), lambda i: (i + 32, 0)),
            pl.BlockSpec((C, D), lambda i: (0, 0)),
            pl.BlockSpec((1, D), lambda i: (0, 0)),
        ],
        out_specs=pl.BlockSpec((2, BR, D), lambda i: (0, i, 0)),
        out_shape=jax.ShapeDtypeStruct((2, H, D), jnp.float32),
    )(h2, g2, cnt2, cnt2, mv, mv, Wo, bo)


def kernel(x, mask, X_features, W1, b1, Wo, bo):
    x2 = x.reshape(N, D).astype(jnp.float32)
    xt3 = (X_features.reshape(N, 8)[:, 2:5].astype(jnp.float32)
           .T.reshape(3, 512, 128))
    mv = mask.reshape(512, 128).astype(jnp.int32)

    h2, seg = _run_a(x2, xt3, mv, W1.astype(jnp.float32),
                     b1.reshape(1, C).astype(jnp.float32))
    g2, cnt = _run_b(h2, seg.reshape(512, 128))
    out = _run_c(h2, g2, cnt.reshape(512, 128), mv,
                 Wo.astype(jnp.float32), bo.reshape(1, D).astype(jnp.float32))
    return out.reshape(N, D).reshape(B, S, D).astype(x.dtype)


# speedup vs baseline: 5.3185x; 1.3784x over previous
import functools

import jax
import jax.numpy as jnp
from jax import lax
from jax.experimental import pallas as pl
from jax.experimental.pallas import tpu as pltpu
from jax.experimental.pallas import tpu_sc as plsc

B, S, D = 16, 4096, 128
C = 64
M = 65536
N = B * S
GRID = 0.01
NTRASH = 1024
MT = M + NTRASH

NSUB = 16
TPW = N // NSUB
CH = 16
H = N // 2
BR = 4096


def _ka_body(xl_ref, xr_ref, xtl_ref, xtr_ref, mvl_ref, mvr_ref,
             w1_ref, b1_ref, h2_ref, seg_ref):
    i = pl.program_id(0)
    w1 = w1_ref[...]
    b1 = b1_ref[...]
    hl = jnp.maximum(jnp.dot(xl_ref[...], w1,
                             preferred_element_type=jnp.float32) + b1, 0.0)
    hr = jnp.maximum(jnp.dot(xr_ref[...], w1,
                             preferred_element_type=jnp.float32) + b1, 0.0)
    h2_ref[...] = jnp.concatenate([hl, hr], axis=1)

    def _seg(xt_blk, mv_blk, tok0):
        v0 = jnp.floor(xt_blk[0] / GRID).astype(jnp.int32)
        v1 = jnp.floor(xt_blk[1] / GRID).astype(jnp.int32)
        v2 = jnp.floor(xt_blk[2] / GRID).astype(jnp.int32)
        tok = (tok0
               + lax.broadcasted_iota(jnp.int32, (32, 128), 0) * 128
               + lax.broadcasted_iota(jnp.int32, (32, 128), 1))
        batch = lax.shift_right_logical(tok, 12)
        hcode = (batch * 73856093 + v0 * 19349663
                 + v1 * 83492791 + v2 * 49979687)
        seg = jnp.bitwise_and(hcode, M - 1)
        trash = M + jnp.bitwise_and(tok, NTRASH - 1)
        return jnp.where(mv_blk > 0, seg, trash)

    seg_ref[0] = _seg(xtl_ref[...], mvl_ref[...], i * BR)
    seg_ref[1] = _seg(xtr_ref[...], mvr_ref[...], H + i * BR)


def _run_a(x2, xt3, mv, W1, b1):
    return pl.pallas_call(
        _ka_body,
        grid=(H // BR,),
        in_specs=[
            pl.BlockSpec((BR, D), lambda i: (i, 0)),
            pl.BlockSpec((BR, D), lambda i: (i + 8, 0)),
            pl.BlockSpec((3, 32, 128), lambda i: (0, i, 0)),
            pl.BlockSpec((3, 32, 128), lambda i: (0, i + 8, 0)),
            pl.BlockSpec((32, 128), lambda i: (i, 0)),
            pl.BlockSpec((32, 128), lambda i: (i + 8, 0)),
            pl.BlockSpec((D, C), lambda i: (0, 0)),
            pl.BlockSpec((1, C), lambda i: (0, 0)),
        ],
        out_specs=[
            pl.BlockSpec((BR, 2 * C), lambda i: (i, 0)),
            pl.BlockSpec((2, 32, 128), lambda i: (0, i, 0)),
        ],
        out_shape=[
            jax.ShapeDtypeStruct((H, 2 * C), jnp.float32),
            jax.ShapeDtypeStruct((2, 256, 128), jnp.int32),
        ],
    )(x2, x2, xt3, xt3, mv, mv, W1, b1)


def _kb_body(h_hbm, seg_hbm, g_hbm, cnt_hbm,
             segv, row0, row1, cbuf, zbuf, z1buf, onesv, sem0, sem1,
             sem_sc, sem_c, table, ctable):
    c = lax.axis_index("c")
    s = lax.axis_index("s")
    tok0 = s * TPW
    rowbase = (s % 8) * TPW
    halfoff = (s // 8) * C
    bufs = (row0, row1)
    sems = (sem0, sem1)
    QCH = 1024
    NQ = TPW // QCH

    zv = jnp.zeros((16,), jnp.float32)

    def _z2(j, _):
        zbuf[j] = zv
        return 0
    lax.fori_loop(0, 256, _z2, 0)

    def _z1(j, _):
        z1buf[pl.ds(j * 16, 16)] = zv
        return 0
    lax.fori_loop(0, 64, _z1, 0)

    ov = jnp.ones((16,), jnp.float32)
    for j in range(8):
        onesv[pl.ds(j * 16, 16)] = ov

    pltpu.sync_copy(seg_hbm.at[pl.ds(s * 32, 32)], segv)

    for r in range(2):
        ch0 = halfoff + c * 32 + r * CH

        def _read(q):
            return pltpu.async_copy(
                h_hbm.at[pl.ds(rowbase + q * QCH, QCH), pl.ds(ch0, CH)],
                bufs[q % 2], sems[q % 2])

        cps = {0: _read(0), 1: _read(1)}

        for j in range(16):
            pltpu.sync_copy(zbuf, table.at[pl.ds(s * 4096 + j * 256, 256)])
        pltpu.sync_copy(zbuf.at[pl.ds(0, 64)],
                        table.at[pl.ds(M + s * 64, 64)])

        if r == 0:
            @pl.when(c == 0)
            def _():
                for j in range(4):
                    pltpu.sync_copy(
                        z1buf, ctable.at[pl.ds(s * 4096 + j * 1024, 1024)])
                pltpu.sync_copy(z1buf.at[pl.ds(0, 64)],
                                ctable.at[pl.ds(M + s * 64, 64)])

        plsc.subcore_barrier()

        for q in range(NQ):
            cps[q].wait()
            for k in range(8):
                pltpu.async_copy(bufs[q % 2].at[pl.ds(k * 128, 128)],
                                 table.at[segv.at[q * 8 + k]], sem_sc,
                                 add=True)
            for k in range(8):
                pltpu.make_async_copy(
                    bufs[q % 2].at[pl.ds(k * 128, 128)],
                    table.at[segv.at[q * 8 + k]], sem_sc).wait()
            if q + 2 < NQ:
                cps[q + 2] = _read(q + 2)

        if r == 0:
            @pl.when(c == 0)
            def _():
                def _cscatter(k, _):
                    pltpu.async_copy(onesv, ctable.at[segv.at[k]], sem_c,
                                     add=True)
                    return 0
                lax.fori_loop(0, 32, _cscatter, 0)

                def _cdrain(k, _):
                    pltpu.make_async_copy(onesv, ctable.at[segv.at[k]],
                                          sem_c).wait()
                    return 0
                lax.fori_loop(0, 32, _cdrain, 0)

        plsc.subcore_barrier()

        wr = {}
        for q in range(NQ):
            if q >= 2:
                wr[q - 2].wait()
            for k in range(8):
                pltpu.async_copy(table.at[segv.at[q * 8 + k]],
                                 bufs[q % 2].at[pl.ds(k * 128, 128)], sem_sc)
            for k in range(8):
                pltpu.make_async_copy(
                    table.at[segv.at[q * 8 + k]],
                    bufs[q % 2].at[pl.ds(k * 128, 128)], sem_sc).wait()
            wr[q] = pltpu.async_copy(
                bufs[q % 2],
                g_hbm.at[pl.ds(rowbase + q * QCH, QCH), pl.ds(ch0, CH)],
                sems[q % 2])
        wr[NQ - 2].wait()
        wr[NQ - 1].wait()

        if r == 0:
            @pl.when(c == 0)
            def _():
                def _cgfire(k, _):
                    pltpu.async_copy(ctable.at[segv.at[k]],
                                     cbuf.at[pl.ds(k * 128, 128)], sem_c)
                    return 0
                lax.fori_loop(0, 32, _cgfire, 0)

                def _cgdrain(k, _):
                    pltpu.make_async_copy(ctable.at[segv.at[k]],
                                          cbuf.at[pl.ds(k * 128, 128)],
                                          sem_c).wait()
                    return 0
                lax.fori_loop(0, 32, _cgdrain, 0)
                pltpu.sync_copy(cbuf, cnt_hbm.at[pl.ds(tok0, TPW)])

        plsc.subcore_barrier()


def _run_b(h2, seg2):
    kb = functools.partial(
        pl.kernel,
        out_type=[
            jax.ShapeDtypeStruct((H, 2 * C), jnp.float32),
            jax.ShapeDtypeStruct((N,), jnp.float32),
        ],
        mesh=plsc.VectorSubcoreMesh(core_axis_name="c", subcore_axis_name="s"),
        compiler_params=pltpu.CompilerParams(use_tc_tiling_on_sc=False),
        scratch_types=[
            pltpu.VMEM((32, 128), jnp.int32),
            pltpu.VMEM((1024, CH), jnp.float32),
            pltpu.VMEM((1024, CH), jnp.float32),
            pltpu.VMEM((TPW,), jnp.float32),
            pltpu.VMEM((256, CH), jnp.float32),
            pltpu.VMEM((1024,), jnp.float32),
            pltpu.VMEM((128,), jnp.float32),
            pltpu.SemaphoreType.DMA,
            pltpu.SemaphoreType.DMA,
            pltpu.SemaphoreType.DMA,
            pltpu.SemaphoreType.DMA,
            pltpu.VMEM_SHARED((MT, CH), jnp.float32),
            pltpu.VMEM_SHARED((MT,), jnp.float32),
        ],
    )(_kb_body)
    return kb(h2, seg2)


def _kc_body(h2_ref, g2_ref, cl_ref, cr_ref, mvl_ref, mvr_ref,
             wo_ref, bo_ref, o_ref):
    wo = wo_ref[...]
    bo = bo_ref[...]
    rsel = (lax.broadcasted_iota(jnp.int32, (BR, 32), 0) // 128
            == lax.broadcasted_iota(jnp.int32, (BR, 32), 1)
            ).astype(jnp.float32)
    lsel = (lax.broadcasted_iota(jnp.int32, (BR, 128), 0) % 128
            == lax.broadcasted_iota(jnp.int32, (BR, 128), 1)
            ).astype(jnp.float32)

    def _cols(blk4):
        expanded = jnp.dot(rsel, blk4, preferred_element_type=jnp.float32)
        return jnp.sum(expanded * lsel, axis=1, keepdims=True)

    h2 = h2_ref[...]
    g2 = g2_ref[...]

    def _half(hs, gs, cnt4, mv4):
        a4 = (mv4 > 0).astype(jnp.float32)
        rc4 = a4 / jnp.maximum(cnt4, 1.0)
        a_col = _cols(a4)
        b_col = _cols(rc4)
        t = hs + gs * b_col
        y = jnp.dot(t, wo, preferred_element_type=jnp.float32) + bo
        return a_col * y

    yl = _half(h2[:, :C], g2[:, :C], cl_ref[...], mvl_ref[...])
    yr = _half(h2[:, C:], g2[:, C:], cr_ref[...], mvr_ref[...])
    o_ref[0] = yl
    o_ref[1] = yr


def _run_c(h2, g2, cnt2, mv, Wo, bo):
    return pl.pallas_call(
        _kc_body,
        grid=(H // BR,),
        in_specs=[
            pl.BlockSpec((BR, 2 * C), lambda i: (i, 0)),
            pl.BlockSpec((BR, 2 * C), lambda i: (i, 0)),
            pl.BlockSpec((32, 128), lambda i: (i, 0)),
            pl.BlockSpec((32, 128), lambda i: (i + 8, 0)),
            pl.BlockSpec((32, 128), lambda i: (i, 0)),
            pl.BlockSpec((32, 128), lambda i: (i + 8, 0)),
            pl.BlockSpec((C, D), lambda i: (0, 0)),
            pl.BlockSpec((1, D), lambda i: (0, 0)),
        ],
        out_specs=pl.BlockSpec((2, BR, D), lambda i: (0, i, 0)),
        out_shape=jax.ShapeDtypeStruct((2, H, D), jnp.float32),
    )(h2, g2, cnt2, cnt2, mv, mv, Wo, bo)


def kernel(x, mask, X_features, W1, b1, Wo, bo):
    x2 = x.reshape(N, D).astype(jnp.float32)
    xt3 = (X_features.reshape(N, 8)[:, 2:5].astype(jnp.float32)
           .T.reshape(3, 512, 128))
    mv = mask.reshape(512, 128).astype(jnp.int32)

    h2, seg = _run_a(x2, xt3, mv, W1.astype(jnp.float32),
                     b1.reshape(1, C).astype(jnp.float32))
    g2, cnt = _run_b(h2, seg.reshape(512, 128))
    out = _run_c(h2, g2, cnt.reshape(512, 128), mv,
                 Wo.astype(jnp.float32), bo.reshape(1, D).astype(jnp.float32))
    return out.reshape(N, D).reshape(B, S, D).astype(x.dtype)
